# baseline (speedup 1.0000x reference)
import functools

import jax
import jax.numpy as jnp
from jax import lax
from jax.experimental import pallas as pl
from jax.experimental.pallas import tpu as pltpu
from jax.experimental.pallas import tpu_sc as plsc

N = 10000
E = 160000
C = 256
HIDDEN = 512

NC = 2
NS = 16
NW = NC * NS
PB = 128
EPW = 5120
NBK = EPW // PB
EPAD = NW * EPW
NP = 10240
RPS = NP // NS
HW = 128
NPH = 7


def _agg_body(xh, srcs, dsts, wcat, zeros,
              outacc,
              src_v, dst_v, wb, msgb, acc,
              gsem0, gsem1, ssem0, ssem1):
    cid = lax.axis_index("c")
    sid = lax.axis_index("s")
    wid = cid * NS + sid

    pltpu.sync_copy(srcs.at[wid], src_v)
    pltpu.sync_copy(dsts.at[wid], dst_v)

    zero16 = jnp.zeros((16,), jnp.float32)
    iota16 = lax.iota(jnp.int32, 16)

    _dnums = lax.GatherDimensionNumbers(
        offset_dims=(), collapsed_slice_dims=(0,), start_index_map=(0,))

    def _bcast(vec16, t):
        idx = jnp.full((16, 1), t, jnp.int32)
        return lax.gather(vec16, idx, _dnums, (1,),
                          mode=lax.GatherScatterMode.PROMISE_IN_BOUNDS)

    row0 = sid * RPS
    gsems = (gsem0, gsem1)
    ssems = (ssem0, ssem1)

    def _zero_acc():
        pltpu.sync_copy(zeros, acc.at[pl.ds(row0, RPS)])

    def _dump(p):
        pltpu.sync_copy(acc.at[pl.ds(row0, RPS)],
                        outacc.at[p * NC + cid, pl.ds(row0, RPS)])

    def _issue_gather(p, g, j):
        half = p // 3
        cf = p - 3 * half
        pltpu.async_copy(xh.at[half].at[src_v.at[g, pl.ds(0, PB // 2)]],
                         msgb.at[j, pl.ds(0, PB // 2)], gsems[j])
        pltpu.async_copy(xh.at[half].at[src_v.at[g, pl.ds(PB // 2, PB // 2)]],
                         msgb.at[j, pl.ds(PB // 2, PB // 2)], gsems[j])
        pltpu.async_copy(wcat.at[cf, wid, g], wb.at[j, pl.ds(0, PB)], gsems[j])

    def _wait_gather(j):
        pltpu.make_async_copy(xh.at[0].at[src_v.at[0, pl.ds(0, PB // 2)]],
                              msgb.at[j, pl.ds(0, PB // 2)], gsems[j]).wait()
        pltpu.make_async_copy(xh.at[0].at[src_v.at[0, pl.ds(0, PB // 2)]],
                              msgb.at[j, pl.ds(PB // 2, PB // 2)], gsems[j]).wait()
        pltpu.make_async_copy(wcat.at[0, 0, 0], wb.at[j, pl.ds(0, PB)], gsems[j]).wait()

    def _compute(j):
        def _grp(gx, _):
            base = gx * 16
            w16 = wb[j, pl.ds(gx * 16, 16)]

            def _e2(t, _):
                e = base + t
                w_b = _bcast(w16, t)
                for r in range(HW // 16):
                    msgb[j, e, pl.ds(r * 16, 16)] = w_b * msgb[j, e, pl.ds(r * 16, 16)]
                return ()
            lax.fori_loop(0, 16, _e2, (), unroll=4)
            return ()
        lax.fori_loop(0, PB // 16, _grp, ())

    def _issue_scatter(g, j):
        pltpu.async_copy(msgb.at[j], acc.at[dst_v.at[g]], ssems[j], add=True)

    def _noop_issue_scatter(g, j):
        pass

    def _wait_scatter(j):
        pltpu.make_async_copy(msgb.at[j], acc.at[pl.ds(0, PB)], ssems[j]).wait()

    def _noop_wait_scatter(j):
        pass

    def _phase(p, _):
        plsc.subcore_barrier()
        _zero_acc()
        plsc.subcore_barrier()

        _issue_gather(p, 0, 0)

        def _quad(q, _):
            g0 = q * 2
            g1 = g0 + 1
            _wait_gather(0)

            @pl.when(g0 >= 1)
            def _():
                _wait_scatter(1)
            _issue_gather(p, g1, 1)
            _compute(0)
            _issue_scatter(g0, 0)
            _wait_gather(1)
            _wait_scatter(0)

            @pl.when(g1 + 1 < NBK)
            def _():
                _issue_gather(p, g1 + 1, 0)
            _compute(1)
            _issue_scatter(g1, 1)
            return ()
        lax.fori_loop(0, NBK // 2, _quad, ())

        _wait_scatter(1)
        plsc.subcore_barrier()
        _dump(p)
        return ()
    lax.fori_loop(0, 6, _phase, ())

    plsc.subcore_barrier()
    _zero_acc()
    plsc.subcore_barrier()

    def _zmsg(e, _):
        for j in range(HW // 16):
            msgb[0, e, pl.ds(j * 16, 16)] = zero16
            msgb[1, e, pl.ds(j * 16, 16)] = zero16
        return ()
    lax.fori_loop(0, PB, _zmsg, ())

    lane0 = iota16 == 0
    lane1 = iota16 == 1
    lane2 = iota16 == 2

    def _issue_w3(g, j):
        for kc in range(3):
            pltpu.async_copy(wcat.at[kc, wid, g], wb.at[j, pl.ds(kc * PB, PB)],
                             gsems[j])

    def _wait_w3(j):
        for kc in range(3):
            pltpu.make_async_copy(wcat.at[0, 0, 0], wb.at[j, pl.ds(kc * PB, PB)],
                                  gsems[j]).wait()

    def _dcompute(j):
        def _dgrp(gx, _):
            base = gx * 16
            wdx16 = wb[j, pl.ds(gx * 16, 16)]
            wdy16 = wb[j, pl.ds(PB + gx * 16, 16)]
            wlap16 = wb[j, pl.ds(2 * PB + gx * 16, 16)]

            def _de(t, _):
                e = base + t
                v = jnp.where(lane0, _bcast(wdx16, t), zero16)
                v = jnp.where(lane1, _bcast(wdy16, t), v)
                v = jnp.where(lane2, _bcast(wlap16, t), v)
                msgb[j, e, pl.ds(0, 16)] = v
                return ()
            lax.fori_loop(0, 16, _de, (), unroll=4)
            return ()
        lax.fori_loop(0, PB // 16, _dgrp, ())

    _issue_w3(0, 0)

    def _dquad(q, _):
        g0 = q * 2
        g1 = g0 + 1
        _wait_w3(0)

        @pl.when(g0 >= 1)
        def _():
            _wait_scatter(1)
        _issue_w3(g1, 1)
        _dcompute(0)
        _issue_scatter(g0, 0)

        _wait_w3(1)
        _wait_scatter(0)

        @pl.when(g1 + 1 < NBK)
        def _():
            _issue_w3(g1 + 1, 0)
        _dcompute(1)
        _issue_scatter(g1, 1)
        return ()
    lax.fori_loop(0, NBK // 2, _dquad, ())

    _wait_scatter(1)
    plsc.subcore_barrier()
    _dump(6)


_agg = functools.partial(
    pl.kernel,
    out_type=[
        jax.ShapeDtypeStruct((NPH * NC, NP, HW), jnp.float32),
    ],
    mesh=plsc.VectorSubcoreMesh(core_axis_name="c", subcore_axis_name="s",
                                num_cores=NC, num_subcores=NS),
    scratch_types=[
        pltpu.VMEM((NBK, PB), jnp.int32),
        pltpu.VMEM((NBK, PB), jnp.int32),
        pltpu.VMEM((2, 3 * PB), jnp.float32),
        pltpu.VMEM((2, PB, HW), jnp.float32),
        pltpu.VMEM_SHARED((NP, HW), jnp.float32),
        pltpu.SemaphoreType.DMA,
        pltpu.SemaphoreType.DMA,
        pltpu.SemaphoreType.DMA,
        pltpu.SemaphoreType.DMA,
    ],
)(_agg_body)


BLK = 128


def _mlp_body(hc_ref, x_ref, acc_ref,
              W1_ref, b1_ref, g1_ref, bt1_ref,
              W2_ref, b2_ref, g2_ref, bt2_ref,
              W3_ref, b3_ref, o_ref):
    h = hc_ref[0, 0]
    xb = x_ref[...]
    acct = acc_ref[...]
    degc = acct[12] + acct[13]

    inv_sqrt2 = 0.7071067811865476

    def gelu(v):
        return 0.5 * v * (1.0 + lax.erf(v * inv_sqrt2))

    def ln(v, g, b):
        mu = jnp.mean(v, axis=-1, keepdims=True)
        var = jnp.mean((v - mu) ** 2, axis=-1, keepdims=True)
        return (v - mu) * lax.rsqrt(var + 1e-5) * g + b

    hs = [h, h, h * h]
    z = xb @ W1_ref[0:C]
    for kc in range(3):
        Y = jnp.concatenate(
            [acct[2 * kc] + acct[2 * kc + 1],
             acct[6 + 2 * kc] + acct[6 + 2 * kc + 1]], axis=1)
        deg = degc[:, kc:kc + 1]
        feat = (Y - deg * xb) * hs[kc]
        z = z + feat @ W1_ref[(kc + 1) * C:(kc + 2) * C]
    z = z + b1_ref[...]
    z = gelu(ln(z, g1_ref[...], bt1_ref[...]))
    z = z @ W2_ref[...] + b2_ref[...]
    z = gelu(ln(z, g2_ref[...], bt2_ref[...]))
    o_ref[...] = z @ W3_ref[...] + b3_ref[...] + xb


def kernel(x, edge_index, coeff_dx, coeff_dy, coeff_lap, h_char,
           W1, b1, g1, bt1, W2, b2, g2, bt2, W3, b3):
    src = edge_index[0]
    dst = edge_index[1]
    pad = EPAD - E
    srcp = jnp.concatenate([src, jnp.zeros((pad,), jnp.int32)]).reshape(NW, NBK, PB)
    dstp = jnp.concatenate([dst, jnp.zeros((pad,), jnp.int32)]).reshape(NW, NBK, PB)
    zpadf = jnp.zeros((pad,), jnp.float32)
    wcat = jnp.stack([
        jnp.concatenate([coeff_dx.reshape(E), zpadf]).reshape(NW, NBK, PB),
        jnp.concatenate([coeff_dy.reshape(E), zpadf]).reshape(NW, NBK, PB),
        jnp.concatenate([coeff_lap.reshape(E), zpadf]).reshape(NW, NBK, PB),
    ])

    xh = jnp.stack([x[:, :HW], x[:, HW:]])
    zeros = jnp.zeros((RPS, HW), jnp.float32)

    (outacc,) = _agg(xh, srcp, dstp, wcat, zeros)

    xp = jnp.pad(x, ((0, NP - N), (0, 0)))
    hc = h_char.reshape(1, 1)

    grid = (NP // BLK,)
    out = pl.pallas_call(
        _mlp_body,
        grid=grid,
        in_specs=[
            pl.BlockSpec(memory_space=pltpu.SMEM),
            pl.BlockSpec((BLK, C), lambda i: (i, 0)),
            pl.BlockSpec((NPH * NC, BLK, HW), lambda i: (0, i, 0)),
            pl.BlockSpec((4 * C, HIDDEN), lambda i: (0, 0)),
            pl.BlockSpec((1, HIDDEN), lambda i: (0, 0)),
            pl.BlockSpec((1, HIDDEN), lambda i: (0, 0)),
            pl.BlockSpec((1, HIDDEN), lambda i: (0, 0)),
            pl.BlockSpec((HIDDEN, C), lambda i: (0, 0)),
            pl.BlockSpec((1, C), lambda i: (0, 0)),
            pl.BlockSpec((1, C), lambda i: (0, 0)),
            pl.BlockSpec((1, C), lambda i: (0, 0)),
            pl.BlockSpec((C, C), lambda i: (0, 0)),
            pl.BlockSpec((1, C), lambda i: (0, 0)),
        ],
        out_specs=pl.BlockSpec((BLK, C), lambda i: (i, 0)),
        out_shape=jax.ShapeDtypeStruct((NP, C), jnp.float32),
    )(hc, xp, outacc,
      W1, b1.reshape(1, HIDDEN), g1.reshape(1, HIDDEN), bt1.reshape(1, HIDDEN),
      W2, b2.reshape(1, C), g2.reshape(1, C), bt2.reshape(1, C),
      W3, b3.reshape(1, C))
    return out[:N]


# speedup vs baseline: 1.0009x; 1.0009x over previous
import functools

import jax
import jax.numpy as jnp
from jax import lax
from jax.experimental import pallas as pl
from jax.experimental.pallas import tpu as pltpu
from jax.experimental.pallas import tpu_sc as plsc

N = 10000
E = 160000
C = 256
HIDDEN = 512

NC = 2
NS = 16
NW = NC * NS
PB = 128
EPW = 5120
NBK = EPW // PB
EPAD = NW * EPW
NP = 10240
RPS = NP // NS
HW = 128
NPH = 7


def _agg_body(xh, srcs, dsts, wcat, zeros,
              outacc,
              src_v, dst_v, wb, msgb, acc,
              gsem0, gsem1, ssem0, ssem1):
    cid = lax.axis_index("c")
    sid = lax.axis_index("s")
    wid = cid * NS + sid

    pltpu.sync_copy(srcs.at[wid], src_v)
    pltpu.sync_copy(dsts.at[wid], dst_v)

    zero16 = jnp.zeros((16,), jnp.float32)
    iota16 = lax.iota(jnp.int32, 16)

    _dnums = lax.GatherDimensionNumbers(
        offset_dims=(), collapsed_slice_dims=(0,), start_index_map=(0,))

    def _bcast(vec16, t):
        idx = jnp.full((16, 1), t, jnp.int32)
        return lax.gather(vec16, idx, _dnums, (1,),
                          mode=lax.GatherScatterMode.PROMISE_IN_BOUNDS)

    row0 = sid * RPS
    gsems = (gsem0, gsem1)
    ssems = (ssem0, ssem1)

    def _zero_acc():
        pltpu.sync_copy(zeros, acc.at[pl.ds(row0, RPS)])

    def _dump(p):
        pltpu.sync_copy(acc.at[pl.ds(row0, RPS)],
                        outacc.at[p * NC + cid, pl.ds(row0, RPS)])

    def _issue_gather(p, g, j):
        half = p // 3
        cf = p - 3 * half
        pltpu.async_copy(xh.at[half].at[src_v.at[g, pl.ds(0, PB // 2)]],
                         msgb.at[j, pl.ds(0, PB // 2)], gsems[j])
        pltpu.async_copy(xh.at[half].at[src_v.at[g, pl.ds(PB // 2, PB // 2)]],
                         msgb.at[j, pl.ds(PB // 2, PB // 2)], gsems[j])
        pltpu.async_copy(wcat.at[cf, wid, g], wb.at[j, pl.ds(0, PB)], gsems[j])

    def _wait_gather(j):
        pltpu.make_async_copy(xh.at[0].at[src_v.at[0, pl.ds(0, PB // 2)]],
                              msgb.at[j, pl.ds(0, PB // 2)], gsems[j]).wait()
        pltpu.make_async_copy(xh.at[0].at[src_v.at[0, pl.ds(0, PB // 2)]],
                              msgb.at[j, pl.ds(PB // 2, PB // 2)], gsems[j]).wait()
        pltpu.make_async_copy(wcat.at[0, 0, 0], wb.at[j, pl.ds(0, PB)], gsems[j]).wait()

    def _compute(j):
        def _grp(gx, _):
            base = gx * 16
            w16 = wb[j, pl.ds(gx * 16, 16)]

            def _e2(t, _):
                e = base + t
                w_b = _bcast(w16, t)
                for r in range(HW // 16):
                    msgb[j, e, pl.ds(r * 16, 16)] = w_b * msgb[j, e, pl.ds(r * 16, 16)]
                return ()
            lax.fori_loop(0, 16, _e2, (), unroll=4)
            return ()
        lax.fori_loop(0, PB // 16, _grp, ())

    def _issue_scatter(g, j):
        pltpu.async_copy(msgb.at[j], acc.at[dst_v.at[g]], ssems[j], add=True)

    def _wait_scatter(j):
        pltpu.make_async_copy(msgb.at[j], acc.at[pl.ds(0, PB)], ssems[j]).wait()

    _issue_gather(0, 0, 0)

    def _phase(p, _):
        plsc.subcore_barrier()
        _zero_acc()
        plsc.subcore_barrier()

        def _quad(q, _):
            g0 = q * 2
            g1 = g0 + 1
            _wait_gather(0)

            @pl.when(g0 >= 1)
            def _():
                _wait_scatter(1)
            _issue_gather(p, g1, 1)
            _compute(0)
            _issue_scatter(g0, 0)
            _wait_gather(1)
            _wait_scatter(0)

            @pl.when(g1 + 1 < NBK)
            def _():
                _issue_gather(p, g1 + 1, 0)
            _compute(1)
            _issue_scatter(g1, 1)
            return ()
        lax.fori_loop(0, NBK // 2, _quad, ())

        _wait_scatter(1)

        @pl.when(p + 1 < 6)
        def _():
            _issue_gather(p + 1, 0, 0)
        plsc.subcore_barrier()
        _dump(p)
        return ()
    lax.fori_loop(0, 6, _phase, ())

    plsc.subcore_barrier()
    _zero_acc()
    plsc.subcore_barrier()

    def _zmsg(e, _):
        for j in range(HW // 16):
            msgb[0, e, pl.ds(j * 16, 16)] = zero16
            msgb[1, e, pl.ds(j * 16, 16)] = zero16
        return ()
    lax.fori_loop(0, PB, _zmsg, ())

    lane0 = iota16 == 0
    lane1 = iota16 == 1
    lane2 = iota16 == 2

    def _issue_w3(g, j):
        for kc in range(3):
            pltpu.async_copy(wcat.at[kc, wid, g], wb.at[j, pl.ds(kc * PB, PB)],
                             gsems[j])

    def _wait_w3(j):
        for kc in range(3):
            pltpu.make_async_copy(wcat.at[0, 0, 0], wb.at[j, pl.ds(kc * PB, PB)],
                                  gsems[j]).wait()

    def _dcompute(j):
        def _dgrp(gx, _):
            base = gx * 16
            wdx16 = wb[j, pl.ds(gx * 16, 16)]
            wdy16 = wb[j, pl.ds(PB + gx * 16, 16)]
            wlap16 = wb[j, pl.ds(2 * PB + gx * 16, 16)]

            def _de(t, _):
                e = base + t
                v = jnp.where(lane0, _bcast(wdx16, t), zero16)
                v = jnp.where(lane1, _bcast(wdy16, t), v)
                v = jnp.where(lane2, _bcast(wlap16, t), v)
                msgb[j, e, pl.ds(0, 16)] = v
                return ()
            lax.fori_loop(0, 16, _de, (), unroll=4)
            return ()
        lax.fori_loop(0, PB // 16, _dgrp, ())

    _issue_w3(0, 0)

    def _dquad(q, _):
        g0 = q * 2
        g1 = g0 + 1
        _wait_w3(0)

        @pl.when(g0 >= 1)
        def _():
            _wait_scatter(1)
        _issue_w3(g1, 1)
        _dcompute(0)
        _issue_scatter(g0, 0)

        _wait_w3(1)
        _wait_scatter(0)

        @pl.when(g1 + 1 < NBK)
        def _():
            _issue_w3(g1 + 1, 0)
        _dcompute(1)
        _issue_scatter(g1, 1)
        return ()
    lax.fori_loop(0, NBK // 2, _dquad, ())

    _wait_scatter(1)
    plsc.subcore_barrier()
    _dump(6)


_agg = functools.partial(
    pl.kernel,
    out_type=[
        jax.ShapeDtypeStruct((NPH * NC, NP, HW), jnp.float32),
    ],
    mesh=plsc.VectorSubcoreMesh(core_axis_name="c", subcore_axis_name="s",
                                num_cores=NC, num_subcores=NS),
    scratch_types=[
        pltpu.VMEM((NBK, PB), jnp.int32),
        pltpu.VMEM((NBK, PB), jnp.int32),
        pltpu.VMEM((2, 3 * PB), jnp.float32),
        pltpu.VMEM((2, PB, HW), jnp.float32),
        pltpu.VMEM_SHARED((NP, HW), jnp.float32),
        pltpu.SemaphoreType.DMA,
        pltpu.SemaphoreType.DMA,
        pltpu.SemaphoreType.DMA,
        pltpu.SemaphoreType.DMA,
    ],
)(_agg_body)


BLK = 128


def _mlp_body(hc_ref, x_ref, acc_ref,
              W1_ref, b1_ref, g1_ref, bt1_ref,
              W2_ref, b2_ref, g2_ref, bt2_ref,
              W3_ref, b3_ref, o_ref):
    h = hc_ref[0, 0]
    xb = x_ref[...]
    acct = acc_ref[...]
    degc = acct[12] + acct[13]

    inv_sqrt2 = 0.7071067811865476

    def gelu(v):
        return 0.5 * v * (1.0 + lax.erf(v * inv_sqrt2))

    def ln(v, g, b):
        mu = jnp.mean(v, axis=-1, keepdims=True)
        var = jnp.mean((v - mu) ** 2, axis=-1, keepdims=True)
        return (v - mu) * lax.rsqrt(var + 1e-5) * g + b

    hs = [h, h, h * h]
    z = xb @ W1_ref[0:C]
    for kc in range(3):
        Y = jnp.concatenate(
            [acct[2 * kc] + acct[2 * kc + 1],
             acct[6 + 2 * kc] + acct[6 + 2 * kc + 1]], axis=1)
        deg = degc[:, kc:kc + 1]
        feat = (Y - deg * xb) * hs[kc]
        z = z + feat @ W1_ref[(kc + 1) * C:(kc + 2) * C]
    z = z + b1_ref[...]
    z = gelu(ln(z, g1_ref[...], bt1_ref[...]))
    z = z @ W2_ref[...] + b2_ref[...]
    z = gelu(ln(z, g2_ref[...], bt2_ref[...]))
    o_ref[...] = z @ W3_ref[...] + b3_ref[...] + xb


def kernel(x, edge_index, coeff_dx, coeff_dy, coeff_lap, h_char,
           W1, b1, g1, bt1, W2, b2, g2, bt2, W3, b3):
    src = edge_index[0]
    dst = edge_index[1]
    pad = EPAD - E
    srcp = jnp.concatenate([src, jnp.zeros((pad,), jnp.int32)]).reshape(NW, NBK, PB)
    dstp = jnp.concatenate([dst, jnp.zeros((pad,), jnp.int32)]).reshape(NW, NBK, PB)
    zpadf = jnp.zeros((pad,), jnp.float32)
    wcat = jnp.stack([
        jnp.concatenate([coeff_dx.reshape(E), zpadf]).reshape(NW, NBK, PB),
        jnp.concatenate([coeff_dy.reshape(E), zpadf]).reshape(NW, NBK, PB),
        jnp.concatenate([coeff_lap.reshape(E), zpadf]).reshape(NW, NBK, PB),
    ])

    xh = jnp.stack([x[:, :HW], x[:, HW:]])
    zeros = jnp.zeros((RPS, HW), jnp.float32)

    (outacc,) = _agg(xh, srcp, dstp, wcat, zeros)

    xp = jnp.pad(x, ((0, NP - N), (0, 0)))
    hc = h_char.reshape(1, 1)

    grid = (NP // BLK,)
    out = pl.pallas_call(
        _mlp_body,
        grid=grid,
        in_specs=[
            pl.BlockSpec(memory_space=pltpu.SMEM),
            pl.BlockSpec((BLK, C), lambda i: (i, 0)),
            pl.BlockSpec((NPH * NC, BLK, HW), lambda i: (0, i, 0)),
            pl.BlockSpec((4 * C, HIDDEN), lambda i: (0, 0)),
            pl.BlockSpec((1, HIDDEN), lambda i: (0, 0)),
            pl.BlockSpec((1, HIDDEN), lambda i: (0, 0)),
            pl.BlockSpec((1, HIDDEN), lambda i: (0, 0)),
            pl.BlockSpec((HIDDEN, C), lambda i: (0, 0)),
            pl.BlockSpec((1, C), lambda i: (0, 0)),
            pl.BlockSpec((1, C), lambda i: (0, 0)),
            pl.BlockSpec((1, C), lambda i: (0, 0)),
            pl.BlockSpec((C, C), lambda i: (0, 0)),
            pl.BlockSpec((1, C), lambda i: (0, 0)),
        ],
        out_specs=pl.BlockSpec((BLK, C), lambda i: (i, 0)),
        out_shape=jax.ShapeDtypeStruct((NP, C), jnp.float32),
    )(hc, xp, outacc,
      W1, b1.reshape(1, HIDDEN), g1.reshape(1, HIDDEN), bt1.reshape(1, HIDDEN),
      W2, b2.reshape(1, C), g2.reshape(1, C), bt2.reshape(1, C),
      W3, b3.reshape(1, C))
    return out[:N]
